# baseline (speedup 1.0000x reference)
import jax
import jax.numpy as jnp
from jax.experimental import pallas as pl


def _outer_sum_kernel(x_ref, yt_ref, o_ref):
    o_ref[...] = x_ref[...][:, :, None] + yt_ref[...][None, :, :]


def kernel(query_size, key_size, x_emb, y_emb):
    q, d = x_emb.shape
    k, _ = y_emb.shape
    x_eff = jnp.take(x_emb, jnp.arange(q) + (query_size - q), axis=0)
    y_eff = jnp.take(y_emb, jnp.arange(k) + (key_size - k), axis=0)

    yt = y_eff.T
    bq = 16
    out3 = pl.pallas_call(
        _outer_sum_kernel,
        grid=(q // bq,),
        in_specs=[
            pl.BlockSpec((bq, d), lambda i: (i, 0)),
            pl.BlockSpec((d, k), lambda i: (0, 0)),
        ],
        out_specs=pl.BlockSpec((bq, d, k), lambda i: (i, 0, 0)),
        out_shape=jax.ShapeDtypeStruct((q, d, k), x_emb.dtype),
    )(x_eff, yt)
    return jnp.transpose(out3, (0, 2, 1))
